# baseline (speedup 1.0000x reference)
import functools

import jax
import jax.numpy as jnp
from jax import lax
from jax.experimental import pallas as pl
from jax.experimental.pallas import tpu as pltpu
from jax.experimental.pallas import tpu_sc as plsc

B = 16384
D = 16
NU = 1000000
CW = 512
NCH = 61
WSPAN = NCH * CW
TAIL0 = 999936
LCAP = B + 128
NBUCK = 64
LCAP2 = B + NBUCK * 16
RING = 32


def _sc_gather_pair(idx, tAT, tBT, tailA, tailB):
    info = plsc.get_sparse_core_info()
    nc = info.num_cores
    assert nc * info.num_subcores == 32

    flat_ty = jax.ShapeDtypeStruct((B * D,), jnp.float32)
    mesh = plsc.VectorSubcoreMesh(core_axis_name="c", subcore_axis_name="s")

    @functools.partial(
        pl.kernel,
        mesh=mesh,
        out_type=(flat_ty, flat_ty),
        scratch_types=[
            pltpu.VMEM((B,), jnp.int32),
            pltpu.VMEM((LCAP,), jnp.int32),
            pltpu.VMEM((LCAP,), jnp.int32),
            pltpu.VMEM((LCAP2,), jnp.int32),
            pltpu.VMEM((LCAP2,), jnp.int32),
            pltpu.VMEM((NBUCK,), jnp.int32),
            pltpu.SMEM((NBUCK,), jnp.int32),
            pltpu.SMEM((NBUCK,), jnp.int32),
            pltpu.SMEM((NBUCK,), jnp.int32),
            pltpu.VMEM((4, 16, CW), jnp.float32),
            pltpu.VMEM((RING * 2 * 16,), jnp.float32),
            pltpu.SemaphoreType.DMA,
            pltpu.SemaphoreType.DMA,
            pltpu.SemaphoreType.DMA,
        ],
        compiler_params=pltpu.CompilerParams(
            use_tc_tiling_on_sc=True, needs_layout_passes=False),
    )
    def k(idx_h, tA_h, tB_h, tailA_h, tailB_h, outA, outB,
          idxv, plist, rlist, plist2, rlist2, hist, offs_s, ends_s, cur_s,
          slab, stag, sem0, sem1, semo):
        wid = lax.axis_index("s") * nc + lax.axis_index("c")
        lo = wid * WSPAN
        hi = jnp.where(wid == 31, NU, lo + WSPAN)
        lanes = lax.iota(jnp.int32, 16)

        def fire(j):
            c0 = lo + j * CW
            for p in (0, 1):
                sem = sem0 if p == 0 else sem1

                @pl.when(lax.rem(j, 2) == p)
                def _():
                    pltpu.async_copy(tA_h.at[:, pl.ds(c0, CW)],
                                     slab.at[2 * p], sem)
                    pltpu.async_copy(tB_h.at[:, pl.ds(c0, CW)],
                                     slab.at[2 * p + 1], sem)

        def fire_tail():
            pltpu.async_copy(tailA_h, slab.at[0, :, pl.ds(0, 128)], sem0)
            pltpu.async_copy(tailB_h, slab.at[1, :, pl.ds(0, 128)], sem0)

        def wait_chunk(k_i):
            for p in (0, 1):
                sem = sem0 if p == 0 else sem1

                @pl.when(lax.rem(k_i, 2) == p)
                def _():
                    @pl.when(k_i <= NCH)
                    def _():
                        for _ in range(2):
                            pltpu.make_async_copy(
                                tA_h.at[:, pl.ds(0, CW)], slab.at[2 * p],
                                sem).wait()

                    @pl.when(k_i == NCH + 1)
                    def _():
                        for _ in range(2):
                            pltpu.make_async_copy(
                                tailA_h, slab.at[2 * p, :, pl.ds(0, 128)],
                                sem).wait()

        def wait_row():
            pltpu.make_async_copy(outA.at[pl.ds(0, 16)],
                                  stag.at[pl.ds(0, 16)], semo).wait()

        fire(0)
        pltpu.sync_copy(idx_h, idxv)

        def strip(s, tot):
            v = idxv[pl.ds(s * 16, 16)]
            pos = lanes + s * 16
            m = (v >= lo) & (v < hi)
            key = pos + jnp.where(m, 0, 1 << 20)
            skey, sval = plsc.sort_key_val(key, v)
            plist[pl.ds(tot, 16)] = skey
            rlist[pl.ds(tot, 16)] = sval
            return tot + plsc.all_reduce_population_count(m)[0]

        total = lax.fori_loop(0, B // 16, strip, jnp.int32(0))
        nstrips = (total + 15) // 16
        lane0 = lanes == 0

        for t in range(NBUCK // 16):
            hist[pl.ds(t * 16, 16)] = jnp.zeros((16,), jnp.int32)

        def hstrip(s, x):
            base = s * 16
            rv = rlist[pl.ds(base, 16)]
            m = (lanes + base) < total
            cidv = jnp.clip((rv - lo) // CW, 0, NCH + 1)
            plsc.addupdate_scatter(hist, [cidv],
                                   jnp.ones((16,), jnp.int32), mask=m)
            return x

        lax.fori_loop(0, nstrips, hstrip, jnp.int32(0))

        acc = jnp.int32(0)
        for t in range(NBUCK // 16):
            cv = hist[pl.ds(t * 16, 16)]
            for kk in range(16):
                c = t * 16 + kk
                offs_s[c] = acc
                cur_s[c] = acc
                ends_s[c] = acc + cv[kk]
                acc = acc + ((cv[kk] + 15) // 16) * 16

        def place(s, x):
            base = s * 16
            pv = plist[pl.ds(base, 16)]
            rv = rlist[pl.ds(base, 16)]
            for kk in range(16):
                p = pv[kk]
                r = rv[kk]
                valid = (base + kk) < total

                @pl.when(valid)
                def _():
                    c = jnp.clip((r - lo) // CW, 0, NCH + 1)
                    cu = cur_s[c]
                    cur_s[c] = cu + 1
                    dest = jnp.full((16,), cu, jnp.int32)
                    plsc.store_scatter(plist2, [dest],
                                       jnp.full((16,), p, jnp.int32),
                                       mask=lane0)
                    plsc.store_scatter(rlist2, [dest],
                                       jnp.full((16,), r, jnp.int32),
                                       mask=lane0)

            return x

        lax.fori_loop(0, nstrips, place, jnp.int32(0))

        def chunk_body(k_i, cnt):
            j = k_i + 1
            last = jnp.where(wid == 31, NCH, NCH - 1)

            @pl.when(j <= last)
            def _():
                fire(j)

            @pl.when((j == NCH + 1) & (wid == 31))
            def _():
                fire_tail()

            wait_chunk(k_i)

            c0 = jnp.where(k_i == NCH + 1, TAIL0, lo + k_i * CW)
            par2 = lax.rem(k_i, 2) * 2
            start = offs_s[k_i]
            end = ends_s[k_i]

            def estrip(jj, cnt):
                base = start + jj * 16
                pv = plist2[pl.ds(base, 16)]
                rv = rlist2[pl.ds(base, 16)]
                c = cnt
                for kk in range(16):
                    p = pv[kk]
                    r = rv[kk]
                    valid = (base + kk) < end

                    @pl.when(valid)
                    def _():
                        @pl.when(c >= RING)
                        def _():
                            wait_row()
                            wait_row()

                        col = jnp.full((16,), r - c0, jnp.int32)
                        vA = plsc.load_gather(slab.at[par2], [lanes, col])
                        vB = plsc.load_gather(slab.at[par2 + 1],
                                              [lanes, col])
                        slot = lax.rem(c, RING) * 32
                        stag[pl.ds(slot, 16)] = vA
                        stag[pl.ds(slot + 16, 16)] = vB
                        off = p * 16
                        pltpu.async_copy(stag.at[pl.ds(slot, 16)],
                                         outA.at[pl.ds(off, 16)], semo)
                        pltpu.async_copy(stag.at[pl.ds(slot + 16, 16)],
                                         outB.at[pl.ds(off, 16)], semo)

                    c = jnp.where(valid, c + 1, c)
                return c

            return lax.fori_loop(0, (end - start + 15) // 16, estrip, cnt)

        niter = jnp.where(wid == 31, NCH + 2, NCH)
        cnt = lax.fori_loop(0, niter, chunk_body, jnp.int32(0))

        ndrain = jnp.minimum(cnt, RING) * 2
        lax.fori_loop(0, ndrain, lambda i, x: (wait_row(), x)[1],
                      jnp.int32(0))

    return k(idx, tAT, tBT, tailA, tailB)


def _tc_dense_body(gu, gi, mu, mi, gWS, gb, W1a, W1b, b1t, W2t, b2t,
                   Wlt, bl, Wf, bf, out):
    f32 = jnp.float32
    g = gu[...] * gi[...]
    gmf = jnp.dot(g, gWS[...], preferred_element_type=f32) + gb[0, 0]
    h = (jnp.dot(mu[...], W1a[...], preferred_element_type=f32)
         + jnp.dot(mi[...], W1b[...], preferred_element_type=f32)
         + b1t[...])
    h = jnp.maximum(h, 0.0)
    h = jnp.dot(h, W2t[...], preferred_element_type=f32) + b2t[...]
    h = jnp.maximum(h, 0.0)
    mlp = jnp.dot(h, Wlt[...], preferred_element_type=f32) + bl[0, 0]
    x = gmf * Wf[0, 0] + mlp * Wf[0, 1] + bf[0, 0]
    out[...] = 1.0 / (1.0 + jnp.exp(-x))


def _tc_dense(gu, gi, mu, mi, gmf_W, gmf_b, W1, b1, W2, b2, Wl, bl, Wf, bf):
    blk = 256
    rows = B // 8
    grid = rows // blk
    eye8 = jnp.eye(8, dtype=jnp.float32)
    gWS = jnp.kron(eye8, gmf_W.T)
    W1a = jnp.kron(eye8, W1[:, :D].T)
    W1b = jnp.kron(eye8, W1[:, D:].T)
    W2t = jnp.kron(eye8, W2.T)
    Wlt = jnp.kron(eye8, Wl.T)
    b1t = jnp.tile(b1, 8).reshape(1, 128)
    b2t = jnp.tile(b2, 8).reshape(1, 64)
    gb = gmf_b.reshape(1, 1)
    blr = bl.reshape(1, 1)
    bfr = bf.reshape(1, 1)

    row_spec = pl.BlockSpec((blk, 128), lambda i: (i, 0))

    def full(a):
        r = a.ndim
        return pl.BlockSpec(a.shape, lambda i, _r=r: (0,) * _r)

    small = [gWS, gb, W1a, W1b, b1t, W2t, b2t, Wlt, blr, Wf, bfr]
    return pl.pallas_call(
        _tc_dense_body,
        grid=(grid,),
        in_specs=[row_spec] * 4 + [full(a) for a in small],
        out_specs=pl.BlockSpec((blk, 8), lambda i: (i, 0)),
        out_shape=jax.ShapeDtypeStruct((rows, 8), jnp.float32),
    )(gu, gi, mu, mi, *small)


def kernel(users, movies, gmf_uemb, gmf_iemb, gmf_W, gmf_b, mlp_uemb,
           mlp_iemb, W1, b1, W2, b2, Wl, bl, Wf, bf):
    zp = jnp.zeros((128 - (NU - TAIL0), D), jnp.float32)

    def tail(t):
        return jnp.concatenate([t[TAIL0:], zp], axis=0).T

    gu_f, mu_f = _sc_gather_pair(users, gmf_uemb.T, mlp_uemb.T,
                                 tail(gmf_uemb), tail(mlp_uemb))
    gi_f, mi_f = _sc_gather_pair(movies, gmf_iemb.T, mlp_iemb.T,
                                 tail(gmf_iemb), tail(mlp_iemb))

    pk = lambda a: a.reshape(B // 8, 128)
    out = _tc_dense(pk(gu_f), pk(gi_f), pk(mu_f), pk(mi_f),
                    gmf_W, gmf_b, W1, b1, W2, b2, Wl, bl, Wf, bf)
    return out.reshape(B, 1)


# speedup vs baseline: 1.1011x; 1.1011x over previous
import functools

import jax
import jax.numpy as jnp
from jax import lax
from jax.experimental import pallas as pl
from jax.experimental.pallas import tpu as pltpu
from jax.experimental.pallas import tpu_sc as plsc

B = 16384
D = 16
NU = 1000000
CW = 1024
BW = 512
NCHF = 31
WSPAN = NCHF * CW
W31LO = 31 * WSPAN
TAIL0 = 999936
LCAP = B + 32
NBUCK = 64
RING = 32
PACK = (1 << 14) - 1


def _sc_gather_pair(idx, tAT, tBT, tailA, tailB):
    info = plsc.get_sparse_core_info()
    nc = info.num_cores
    assert nc * info.num_subcores == 32

    flat_ty = jax.ShapeDtypeStruct((B * D,), jnp.float32)
    mesh = plsc.VectorSubcoreMesh(core_axis_name="c", subcore_axis_name="s")

    @functools.partial(
        pl.kernel,
        mesh=mesh,
        out_type=(flat_ty, flat_ty),
        scratch_types=[
            pltpu.VMEM((B,), jnp.int32),
            pltpu.VMEM((LCAP,), jnp.int32),
            pltpu.VMEM((LCAP,), jnp.int32),
            pltpu.VMEM((NBUCK,), jnp.int32),
            pltpu.SMEM((NBUCK,), jnp.int32),
            pltpu.SMEM((NBUCK,), jnp.int32),
            pltpu.SMEM((NBUCK,), jnp.int32),
            pltpu.VMEM((4, 16, CW), jnp.float32),
            pltpu.VMEM((RING * 2 * 16,), jnp.float32),
            pltpu.SemaphoreType.DMA,
            pltpu.SemaphoreType.DMA,
            pltpu.SemaphoreType.DMA,
        ],
        compiler_params=pltpu.CompilerParams(
            use_tc_tiling_on_sc=True, needs_layout_passes=False),
    )
    def k(idx_h, tA_h, tB_h, tailA_h, tailB_h, outA, outB,
          idxv, plist, plist2, hist, offs_s, ends_s, cur_s,
          slab, stag, sem0, sem1, semo):
        wid = lax.axis_index("s") * nc + lax.axis_index("c")
        w31 = wid == 31
        lo = wid * WSPAN
        hi = jnp.where(w31, NU, lo + WSPAN)
        nfull = jnp.where(w31, 14, NCHF - 1)
        lanes = lax.iota(jnp.int32, 16)

        def fire(j):
            c0 = lo + j * CW
            for p in (0, 1):
                sem = sem0 if p == 0 else sem1

                @pl.when(lax.rem(j, 2) == p)
                def _():
                    pltpu.async_copy(tA_h.at[:, pl.ds(c0, CW)],
                                     slab.at[2 * p], sem)
                    pltpu.async_copy(tB_h.at[:, pl.ds(c0, CW)],
                                     slab.at[2 * p + 1], sem)

        def fire_512():
            c0 = W31LO + 15 * CW
            pltpu.async_copy(tA_h.at[:, pl.ds(c0, BW)],
                             slab.at[2, :, pl.ds(0, BW)], sem1)
            pltpu.async_copy(tB_h.at[:, pl.ds(c0, BW)],
                             slab.at[3, :, pl.ds(0, BW)], sem1)

        def fire_tail():
            pltpu.async_copy(tailA_h, slab.at[0, :, pl.ds(0, 128)], sem0)
            pltpu.async_copy(tailB_h, slab.at[1, :, pl.ds(0, 128)], sem0)

        def wait_chunk(k_i):
            for p in (0, 1):
                sem = sem0 if p == 0 else sem1

                @pl.when(lax.rem(k_i, 2) == p)
                def _():
                    @pl.when(k_i <= nfull)
                    def _():
                        for _ in range(2):
                            pltpu.make_async_copy(
                                tA_h.at[:, pl.ds(0, CW)], slab.at[2 * p],
                                sem).wait()

                    @pl.when(w31 & (k_i == 15))
                    def _():
                        for _ in range(2):
                            pltpu.make_async_copy(
                                tA_h.at[:, pl.ds(0, BW)],
                                slab.at[2 * p, :, pl.ds(0, BW)], sem).wait()

                    @pl.when(w31 & (k_i == 16))
                    def _():
                        for _ in range(2):
                            pltpu.make_async_copy(
                                tailA_h, slab.at[2 * p, :, pl.ds(0, 128)],
                                sem).wait()

        def wait_row():
            pltpu.make_async_copy(outA.at[pl.ds(0, 16)],
                                  stag.at[pl.ds(0, 16)], semo).wait()

        fire(0)
        pltpu.sync_copy(idx_h, idxv)

        def strip(s, tot):
            v = idxv[pl.ds(s * 16, 16)]
            pos = lanes + s * 16
            m = (v >= lo) & (v < hi)
            packed = ((v - lo) << 14) | pos
            key = jnp.where(m, packed, jnp.int32(1 << 30))
            skey, _ = plsc.sort_key_val(key, key)
            plist[pl.ds(tot, 16)] = skey
            return tot + plsc.all_reduce_population_count(m)[0]

        total = lax.fori_loop(0, B // 16, strip, jnp.int32(0))
        nstrips = (total + 15) // 16
        lane0 = lanes == 0

        for t in range(NBUCK // 16):
            hist[pl.ds(t * 16, 16)] = jnp.zeros((16,), jnp.int32)

        def hstrip(s, x):
            base = s * 16
            ev = plist[pl.ds(base, 16)]
            m = (lanes + base) < total
            plsc.addupdate_scatter(hist, [jnp.clip(ev >> 23, 0, NBUCK - 1)],
                                   jnp.ones((16,), jnp.int32), mask=m)
            return x

        lax.fori_loop(0, nstrips, hstrip, jnp.int32(0))

        acc = jnp.int32(0)
        for t in range(NBUCK // 16):
            cv = hist[pl.ds(t * 16, 16)]
            for kk in range(16):
                c = t * 16 + kk
                offs_s[c] = acc
                cur_s[c] = acc
                acc = acc + cv[kk]
                ends_s[c] = acc

        def place(s, x):
            base = s * 16
            ev = plist[pl.ds(base, 16)]
            for kk in range(16):
                e = ev[kk]
                valid = (base + kk) < total

                @pl.when(valid)
                def _():
                    c = e >> 23
                    cu = cur_s[c]
                    cur_s[c] = cu + 1
                    plsc.store_scatter(plist2,
                                       [jnp.full((16,), cu, jnp.int32)],
                                       jnp.full((16,), e, jnp.int32),
                                       mask=lane0)

            return x

        lax.fori_loop(0, nstrips, place, jnp.int32(0))

        def chunk_body(k_i, cnt):
            j = k_i + 1

            @pl.when(j <= nfull)
            def _():
                fire(j)

            @pl.when(w31 & (j == 15))
            def _():
                fire_512()

            @pl.when(w31 & (j == 16))
            def _():
                fire_tail()

            wait_chunk(k_i)

            istail = w31 & (k_i == 16)
            is512 = w31 & (k_i == 15)
            colbase = jnp.where(istail, TAIL0 - W31LO, k_i * CW)
            b0 = jnp.where(istail, 31, jnp.where(is512, 30, 2 * k_i))
            b1 = jnp.where(istail | is512, b0, 2 * k_i + 1)
            par2 = lax.rem(k_i, 2) * 2
            start = offs_s[b0]
            end = ends_s[b1]

            def estrip(jj, cnt):
                base = start + jj * 16
                ev = plist2[pl.ds(base, 16)]
                c = cnt
                for kk in range(16):
                    e = ev[kk]
                    valid = (base + kk) < end

                    @pl.when(valid)
                    def _():
                        @pl.when(c >= RING)
                        def _():
                            wait_row()
                            wait_row()

                        col = jnp.full((16,), (e >> 14) - colbase,
                                       jnp.int32)
                        vA = plsc.load_gather(slab.at[par2], [lanes, col])
                        vB = plsc.load_gather(slab.at[par2 + 1],
                                              [lanes, col])
                        slot = lax.rem(c, RING) * 32
                        stag[pl.ds(slot, 16)] = vA
                        stag[pl.ds(slot + 16, 16)] = vB
                        off = (e & PACK) * 16
                        pltpu.async_copy(stag.at[pl.ds(slot, 16)],
                                         outA.at[pl.ds(off, 16)], semo)
                        pltpu.async_copy(stag.at[pl.ds(slot + 16, 16)],
                                         outB.at[pl.ds(off, 16)], semo)

                    c = jnp.where(valid, c + 1, c)
                return c

            return lax.fori_loop(0, (end - start + 15) // 16, estrip, cnt)

        niter = jnp.where(w31, 17, NCHF)
        cnt = lax.fori_loop(0, niter, chunk_body, jnp.int32(0))

        ndrain = jnp.minimum(cnt, RING) * 2
        lax.fori_loop(0, ndrain, lambda i, x: (wait_row(), x)[1],
                      jnp.int32(0))

    return k(idx, tAT, tBT, tailA, tailB)


def _tc_dense_body(gu, gi, mu, mi, gWS, gb, W1a, W1b, b1t, W2t, b2t,
                   Wlt, bl, Wf, bf, out):
    f32 = jnp.float32
    g = gu[...] * gi[...]
    gmf = jnp.dot(g, gWS[...], preferred_element_type=f32) + gb[0, 0]
    h = (jnp.dot(mu[...], W1a[...], preferred_element_type=f32)
         + jnp.dot(mi[...], W1b[...], preferred_element_type=f32)
         + b1t[...])
    h = jnp.maximum(h, 0.0)
    h = jnp.dot(h, W2t[...], preferred_element_type=f32) + b2t[...]
    h = jnp.maximum(h, 0.0)
    mlp = jnp.dot(h, Wlt[...], preferred_element_type=f32) + bl[0, 0]
    x = gmf * Wf[0, 0] + mlp * Wf[0, 1] + bf[0, 0]
    out[...] = 1.0 / (1.0 + jnp.exp(-x))


def _tc_dense(gu, gi, mu, mi, gmf_W, gmf_b, W1, b1, W2, b2, Wl, bl, Wf, bf):
    blk = 256
    rows = B // 8
    grid = rows // blk
    eye8 = jnp.eye(8, dtype=jnp.float32)
    gWS = jnp.kron(eye8, gmf_W.T)
    W1a = jnp.kron(eye8, W1[:, :D].T)
    W1b = jnp.kron(eye8, W1[:, D:].T)
    W2t = jnp.kron(eye8, W2.T)
    Wlt = jnp.kron(eye8, Wl.T)
    b1t = jnp.tile(b1, 8).reshape(1, 128)
    b2t = jnp.tile(b2, 8).reshape(1, 64)
    gb = gmf_b.reshape(1, 1)
    blr = bl.reshape(1, 1)
    bfr = bf.reshape(1, 1)

    row_spec = pl.BlockSpec((blk, 128), lambda i: (i, 0))

    def full(a):
        r = a.ndim
        return pl.BlockSpec(a.shape, lambda i, _r=r: (0,) * _r)

    small = [gWS, gb, W1a, W1b, b1t, W2t, b2t, Wlt, blr, Wf, bfr]
    return pl.pallas_call(
        _tc_dense_body,
        grid=(grid,),
        in_specs=[row_spec] * 4 + [full(a) for a in small],
        out_specs=pl.BlockSpec((blk, 8), lambda i: (i, 0)),
        out_shape=jax.ShapeDtypeStruct((rows, 8), jnp.float32),
    )(gu, gi, mu, mi, *small)


def kernel(users, movies, gmf_uemb, gmf_iemb, gmf_W, gmf_b, mlp_uemb,
           mlp_iemb, W1, b1, W2, b2, Wl, bl, Wf, bf):
    zp = jnp.zeros((128 - (NU - TAIL0), D), jnp.float32)

    def tail(t):
        return jnp.concatenate([t[TAIL0:], zp], axis=0).T

    gu_f, mu_f = _sc_gather_pair(users, gmf_uemb.T, mlp_uemb.T,
                                 tail(gmf_uemb), tail(mlp_uemb))
    gi_f, mi_f = _sc_gather_pair(movies, gmf_iemb.T, mlp_iemb.T,
                                 tail(gmf_iemb), tail(mlp_iemb))

    pk = lambda a: a.reshape(B // 8, 128)
    out = _tc_dense(pk(gu_f), pk(gi_f), pk(mu_f), pk(mi_f),
                    gmf_W, gmf_b, W1, b1, W2, b2, Wl, bl, Wf, bf)
    return out.reshape(B, 1)
